# Initial kernel scaffold; baseline (speedup 1.0000x reference)
import jax
import jax.numpy as jnp
from jax.experimental import pallas as pl


def kernel(x, edge_index, W1, b1, W2, b2, W3, b3):
    raise NotImplementedError("write your pallas kernel here")


# speedup vs baseline: 12.7647x; 12.7647x over previous
import functools

import jax
import jax.numpy as jnp
from jax import lax
from jax.experimental import pallas as pl
from jax.experimental.pallas import tpu as pltpu
from jax.experimental.pallas import tpu_sc as plsc

N_NODES = 10000
N_EDGES = 320000
D = 128

NC = 2
NS = 16
NW = NC * NS
NP = 10240
ROWS_PER_TILE = NP // NS
EW = N_EDGES // NW
C = 80
NCH = EW // C
ZR = 64

_mesh = plsc.VectorSubcoreMesh(core_axis_name="c", subcore_axis_name="s")


def _worker_id():
    return lax.axis_index("s") * NC + lax.axis_index("c")


def _zero_fill(zbuf, n_rows, n_cols):
    z = jnp.zeros((16,), jnp.float32)
    per_row = n_cols // 16

    def body(i, _):
        r = i // per_row
        c = (i % per_row) * 16
        zbuf[r, pl.ds(c, 16)] = z
        return 0

    lax.fori_loop(0, n_rows * per_row, body, 0)


@functools.partial(
    pl.kernel,
    out_type=jax.ShapeDtypeStruct((NC * NP,), jnp.float32),
    mesh=_mesh,
    scratch_types=[
        pltpu.VMEM_SHARED((NP,), jnp.float32),
        pltpu.VMEM((C,), jnp.int32),
        pltpu.VMEM((C,), jnp.float32),
        pltpu.VMEM((ROWS_PER_TILE,), jnp.float32),
    ],
)
def _deg_kernel(dst_hbm, out_hbm, acc_sh, dst_v, ones_v, zbuf):
    cid = lax.axis_index("c")
    sid = lax.axis_index("s")
    wid = sid * NC + cid

    one = jnp.ones((16,), jnp.float32)
    z = jnp.zeros((16,), jnp.float32)

    def fill_ones(i, _):
        ones_v[pl.ds(i * 16, 16)] = one
        return 0

    lax.fori_loop(0, C // 16, fill_ones, 0)

    def fill_z(i, _):
        zbuf[pl.ds(i * 16, 16)] = z
        return 0

    lax.fori_loop(0, ROWS_PER_TILE // 16, fill_z, 0)
    pltpu.sync_copy(zbuf, acc_sh.at[pl.ds(sid * ROWS_PER_TILE, ROWS_PER_TILE)])
    plsc.subcore_barrier()

    base = wid * EW

    def body(j, _):
        pltpu.sync_copy(dst_hbm.at[pl.ds(base + j * C, C)], dst_v)
        pltpu.sync_copy(ones_v, acc_sh.at[dst_v], add=True)
        return 0

    lax.fori_loop(0, NCH, body, 0)
    plsc.subcore_barrier()
    off = sid * ROWS_PER_TILE
    pltpu.sync_copy(acc_sh.at[pl.ds(off, ROWS_PER_TILE)],
                    out_hbm.at[pl.ds(cid * NP + off, ROWS_PER_TILE)])


@functools.partial(
    pl.kernel,
    out_type=jax.ShapeDtypeStruct((NC * NP, D), jnp.float32),
    mesh=_mesh,
    scratch_types=[
        pltpu.VMEM_SHARED((NP, D), jnp.float32),
        pltpu.VMEM((C,), jnp.int32),
        pltpu.VMEM((C,), jnp.int32),
        pltpu.VMEM((C, D), jnp.float32),
        pltpu.VMEM((ZR, D), jnp.float32),
        pltpu.SemaphoreType.DMA,
    ],
)
def _agg_kernel(src_hbm, dst_hbm, h_hbm, out_hbm,
                acc_sh, src_v, dst_v, rows_v, zbuf, sem):
    cid = lax.axis_index("c")
    sid = lax.axis_index("s")
    wid = sid * NC + cid

    _zero_fill(zbuf, ZR, D)
    row0 = sid * ROWS_PER_TILE

    def zcopy(k, _):
        pltpu.sync_copy(zbuf, acc_sh.at[pl.ds(row0 + k * ZR, ZR)])
        return 0

    lax.fori_loop(0, ROWS_PER_TILE // ZR, zcopy, 0)
    plsc.subcore_barrier()

    base = wid * EW

    def body(j, _):
        pltpu.sync_copy(src_hbm.at[pl.ds(base + j * C, C)], src_v)
        pltpu.sync_copy(dst_hbm.at[pl.ds(base + j * C, C)], dst_v)
        pltpu.async_copy(h_hbm.at[src_v], rows_v, sem).wait()
        pltpu.sync_copy(rows_v, acc_sh.at[dst_v], add=True)
        return 0

    lax.fori_loop(0, NCH, body, 0)
    plsc.subcore_barrier()
    pltpu.sync_copy(acc_sh.at[pl.ds(row0, ROWS_PER_TILE)],
                    out_hbm.at[pl.ds(cid * NP + row0, ROWS_PER_TILE)])


@functools.partial(
    pl.kernel,
    out_type=jax.ShapeDtypeStruct((NC * NP,), jnp.float32),
    mesh=_mesh,
    scratch_types=[
        pltpu.VMEM_SHARED((NP,), jnp.float32),
        pltpu.VMEM((C,), jnp.int32),
        pltpu.VMEM((C,), jnp.int32),
        pltpu.VMEM((C,), jnp.float32),
        pltpu.VMEM((ROWS_PER_TILE,), jnp.float32),
        pltpu.SemaphoreType.DMA,
    ],
)
def _agg1_kernel(src_hbm, dst_hbm, h_hbm, out_hbm,
                 acc_sh, src_v, dst_v, vals_v, zbuf, sem):
    cid = lax.axis_index("c")
    sid = lax.axis_index("s")
    wid = sid * NC + cid

    z = jnp.zeros((16,), jnp.float32)

    def fill_z(i, _):
        zbuf[pl.ds(i * 16, 16)] = z
        return 0

    lax.fori_loop(0, ROWS_PER_TILE // 16, fill_z, 0)
    pltpu.sync_copy(zbuf, acc_sh.at[pl.ds(sid * ROWS_PER_TILE, ROWS_PER_TILE)])
    plsc.subcore_barrier()

    base = wid * EW

    def body(j, _):
        pltpu.sync_copy(src_hbm.at[pl.ds(base + j * C, C)], src_v)
        pltpu.sync_copy(dst_hbm.at[pl.ds(base + j * C, C)], dst_v)
        pltpu.async_copy(h_hbm.at[src_v], vals_v, sem).wait()
        pltpu.sync_copy(vals_v, acc_sh.at[dst_v], add=True)
        return 0

    lax.fori_loop(0, NCH, body, 0)
    plsc.subcore_barrier()
    off = sid * ROWS_PER_TILE
    pltpu.sync_copy(acc_sh.at[pl.ds(off, ROWS_PER_TILE)],
                    out_hbm.at[pl.ds(cid * NP + off, ROWS_PER_TILE)])


def _dinv_body(dp_ref, o_ref):
    deg = dp_ref[0] + dp_ref[1] + 1.0
    o_ref[...] = lax.rsqrt(deg)


def _mm_scale_body(x_ref, w_ref, dcol_ref, o_ref):
    o_ref[...] = jnp.dot(x_ref[...], w_ref[...],
                         preferred_element_type=jnp.float32) * dcol_ref[...]


def _layer_body(a0_ref, a1_ref, hp_ref, dcol_ref, b_ref, w_ref, o_ref):
    s = (a0_ref[...] + a1_ref[...] + hp_ref[...]) * dcol_ref[...] + b_ref[...]
    h = jnp.maximum(s, 0.0)
    o_ref[...] = jnp.dot(h, w_ref[...],
                         preferred_element_type=jnp.float32) * dcol_ref[...]


def _final_body(a0_ref, a1_ref, hp_ref, dcol_ref, b_ref, o_ref):
    o_ref[...] = ((a0_ref[...] + a1_ref[...] + hp_ref[...]) * dcol_ref[...]
                  + b_ref[...])


def _tc(body, out_shape, *args):
    return pl.pallas_call(
        body, out_shape=jax.ShapeDtypeStruct(out_shape, jnp.float32))(*args)


def kernel(x, edge_index, W1, b1, W2, b2, W3, b3):
    src = edge_index[0].astype(jnp.int32)
    dst = edge_index[1].astype(jnp.int32)

    deg_parts = _deg_kernel(dst)
    dinv2d = _tc(_dinv_body, (NP // D, D), deg_parts.reshape(NC, NP // D, D))
    dinv_col = dinv2d.reshape(NP)[:N_NODES].reshape(N_NODES, 1)

    b1r = b1.reshape(1, D)
    b2r = b2.reshape(1, D)
    b3r = b3.reshape(1, 1)

    h1p = _tc(_mm_scale_body, (N_NODES, D), x, W1, dinv_col)
    agg1 = _agg_kernel(src, dst, h1p).reshape(NC, NP, D)
    h2p = _tc(_layer_body, (N_NODES, D),
              agg1[0, :N_NODES], agg1[1, :N_NODES], h1p, dinv_col, b1r, W2)
    agg2 = _agg_kernel(src, dst, h2p).reshape(NC, NP, D)
    h3p = _tc(_layer_body, (N_NODES, 1),
              agg2[0, :N_NODES], agg2[1, :N_NODES], h2p, dinv_col, b2r, W3)
    agg3 = _agg1_kernel(src, dst, h3p.reshape(N_NODES)).reshape(NC, NP)
    out = _tc(_final_body, (N_NODES, 1),
              agg3[0, :N_NODES].reshape(N_NODES, 1),
              agg3[1, :N_NODES].reshape(N_NODES, 1),
              h3p, dinv_col, b3r)
    return out
